# baseline (speedup 1.0000x reference)
import dataclasses
import functools

import jax
import jax.numpy as jnp
from jax import lax
from jax.experimental import pallas as pl
from jax.experimental.pallas import tpu as pltpu
from jax.experimental.pallas import tpu_sc as plsc

NC = 2
NS = 16
NW = NC * NS
CH = 128
PACK = 1 << 14
CORE0_FRAC = 0.51


def _sc_segment_sum(x_var, packed3, z_acc, z_cnt,
                    nca, ncb, n_acc, rows_per_sub, H):
    ncmax = -(-max(nca, ncb) // 8) * 8
    win = -(-ncmax // 16) * 8
    mesh = plsc.VectorSubcoreMesh(core_axis_name="c", subcore_axis_name="s",
                                  num_cores=NC, num_subcores=NS)
    cp = pltpu.CompilerParams()
    if "needs_layout_passes" in pltpu.CompilerParams.__dataclass_fields__:
        cp = dataclasses.replace(cp, needs_layout_passes=False)

    @functools.partial(
        pl.kernel,
        compiler_params=cp,
        out_type=(
            jax.ShapeDtypeStruct((NC, n_acc, H), jnp.float32),
            jax.ShapeDtypeStruct((NW, n_acc), jnp.float32),
        ),
        mesh=mesh,
        scratch_types=[
            pltpu.VMEM((win, CH), jnp.int32),
            pltpu.VMEM((2, CH), jnp.int32),
            pltpu.VMEM((2, CH), jnp.int32),
            pltpu.VMEM((2, CH, H), jnp.float32),
            pltpu.VMEM((n_acc,), jnp.float32),
            pltpu.VMEM_SHARED((n_acc, H), jnp.float32),
            pltpu.SemaphoreType.DMA,
            pltpu.SemaphoreType.DMA,
        ],
    )
    def sc_kernel(x_hbm, pk_hbm, zacc_hbm, zcnt_hbm, acc_hbm, cnt_hbm,
                  pk_v, src_r, dst_r, rows_v, cnt_v, acc_sh, sem0, sem1):
        cid = lax.axis_index("c")
        sid = lax.axis_index("s")
        wid = sid * NC + cid
        nc = jnp.where(cid == 0, nca, ncb)
        base = pl.multiple_of(cid * nca, 8)
        pltpu.sync_copy(zcnt_hbm, cnt_v)
        row0 = sid * rows_per_sub
        pltpu.sync_copy(zacc_hbm, acc_sh.at[pl.ds(row0, rows_per_sub)])
        plsc.subcore_barrier()

        ones_reg = jnp.ones((16,), jnp.float32)
        sems = (sem0, sem1)

        def unpack(j, ring):
            @pl.loop(0, CH // 16)
            def _(k):
                p = pk_v[j, pl.ds(k * 16, 16)]
                d16 = lax.bitwise_and(p, PACK - 1)
                src_r[ring, pl.ds(k * 16, 16)] = lax.shift_right_logical(p, 14)
                dst_r[ring, pl.ds(k * 16, 16)] = d16
                plsc.addupdate_scatter(cnt_v, [d16], ones_reg)

        def fire(buf):
            pltpu.async_copy(x_hbm.at[src_r.at[buf]], rows_v.at[buf],
                             sems[buf])

        def drain(buf):
            pltpu.make_async_copy(x_hbm.at[src_r.at[buf]], rows_v.at[buf],
                                  sems[buf]).wait()
            pltpu.sync_copy(rows_v.at[buf], acc_sh.at[dst_r.at[buf]],
                            add=True)

        def phase(w, n):
            pltpu.sync_copy(pk_hbm.at[sid, pl.ds(base + w * win, win)], pk_v)

            @pl.when(n >= 1)
            def _():
                unpack(0, 0)
                fire(0)

            @pl.when(n >= 2)
            def _():
                unpack(1, 1)
                fire(1)

            @pl.loop(0, n, step=2)
            def _(j):
                drain(0)

                @pl.when(j + 2 < n)
                def _():
                    unpack(j + 2, 0)
                    fire(0)

                @pl.when(j + 1 < n)
                def _():
                    drain(1)

                @pl.when(j + 3 < n)
                def _():
                    unpack(j + 3, 1)
                    fire(1)

        n0 = jnp.minimum(nc, win)
        phase(0, n0)

        @pl.when(nc > win)
        def _():
            phase(1, nc - win)

        plsc.subcore_barrier()
        pltpu.sync_copy(acc_sh.at[pl.ds(row0, rows_per_sub)],
                        acc_hbm.at[cid, pl.ds(row0, rows_per_sub)])
        pltpu.sync_copy(cnt_v, cnt_hbm.at[wid])

    return sc_kernel(x_var, packed3, z_acc, z_cnt)


def _tail_body(acc_ref, cnt_ref, clue_ref, wm_ref, bm_ref, wua_ref, wc_ref,
               bu_ref, g_ref, be_ref, o_ref):
    A = acc_ref[0] + acc_ref[1]
    cnt = jnp.sum(cnt_ref[...], axis=1, keepdims=True)
    m = lax.dot_general(A, wm_ref[...], (((1,), (0,)), ((), ())),
                        precision=lax.Precision.HIGHEST)
    agg = (m + cnt * bm_ref[...]) / (cnt + 1e-6)
    u = lax.dot_general(agg, wua_ref[...], (((1,), (0,)), ((), ())),
                        precision=lax.Precision.HIGHEST)
    u = u + clue_ref[...] * wc_ref[...] + bu_ref[...]
    u = jnp.maximum(u, 0.0)
    mu = jnp.mean(u, axis=1, keepdims=True)
    var = jnp.mean((u - mu) ** 2, axis=1, keepdims=True)
    o_ref[...] = (u - mu) * lax.rsqrt(var + 1e-5) * g_ref[...] + be_ref[...]


def kernel(x_var, edge_index, clue_values, num_con,
           W_msg, b_msg, W_upd, b_upd, gamma, beta):
    N_var, H = x_var.shape
    N_con = clue_values.shape[0]
    E = edge_index.shape[1]
    src = edge_index[0].astype(jnp.int32)
    dst = edge_index[1].astype(jnp.int32)

    packed = src * PACK + dst
    total_chunks = -(-E // CH)
    per_sid = -(-total_chunks // NS)
    cap = per_sid * NS * CH
    if cap > E:
        packed = jnp.concatenate(
            [packed, jnp.full((cap - E,), N_con, jnp.int32)])
    nca = int(round(per_sid * CORE0_FRAC / 8)) * 8
    ncb = per_sid - nca
    ncmax = -(-max(nca, ncb) // 8) * 8
    win = -(-ncmax // 16) * 8
    dim2 = max(per_sid, nca + 2 * win)
    packed3 = packed.reshape(NS, per_sid, CH)
    if dim2 > per_sid:
        packed3 = jnp.concatenate(
            [packed3, jnp.full((NS, dim2 - per_sid, CH), N_con, jnp.int32)],
            axis=1)

    rows_per_sub = (-(-(N_con + 1) // NS) + 7) // 8 * 8
    n_acc = rows_per_sub * NS

    z_acc = jnp.zeros((rows_per_sub, H), jnp.float32)
    z_cnt = jnp.zeros((n_acc,), jnp.float32)

    acc, cnt = _sc_segment_sum(x_var, packed3, z_acc, z_cnt,
                               nca, ncb, n_acc, rows_per_sub, H)
    cnt_t = cnt.T

    delta = (jnp.asarray(num_con) - N_con).astype(jnp.float32)
    beta_eff = (beta + delta).reshape(1, H)

    BLK = 1000
    grid = -(-N_con // BLK)
    out = pl.pallas_call(
        _tail_body,
        grid=(grid,),
        in_specs=[
            pl.BlockSpec((NC, BLK, H), lambda i: (0, i, 0)),
            pl.BlockSpec((BLK, NW), lambda i: (i, 0)),
            pl.BlockSpec((BLK, 1), lambda i: (i, 0)),
            pl.BlockSpec((H, H), lambda i: (0, 0)),
            pl.BlockSpec((1, H), lambda i: (0, 0)),
            pl.BlockSpec((H, H), lambda i: (0, 0)),
            pl.BlockSpec((1, H), lambda i: (0, 0)),
            pl.BlockSpec((1, H), lambda i: (0, 0)),
            pl.BlockSpec((1, H), lambda i: (0, 0)),
            pl.BlockSpec((1, H), lambda i: (0, 0)),
        ],
        out_specs=pl.BlockSpec((BLK, H), lambda i: (i, 0)),
        out_shape=jax.ShapeDtypeStruct((N_con, H), jnp.float32),
    )(acc, cnt_t, clue_values.reshape(N_con, 1), W_msg, b_msg.reshape(1, H),
      W_upd[:H], W_upd[H:H + 1], b_upd.reshape(1, H), gamma.reshape(1, H),
      beta_eff)
    return out


# speedup vs baseline: 1.0338x; 1.0338x over previous
import dataclasses
import functools

import jax
import jax.numpy as jnp
from jax import lax
from jax.experimental import pallas as pl
from jax.experimental.pallas import tpu as pltpu
from jax.experimental.pallas import tpu_sc as plsc

NC = 2
NS = 16
NW = NC * NS
CH = 128
PACK = 1 << 14
CORE0_FRAC = 0.55


def _sc_segment_sum(x_var, packed3, z_acc, z_cnt,
                    nca, ncb, n_acc, rows_per_sub, H):
    ncmax = -(-max(nca, ncb) // 8) * 8
    win = -(-ncmax // 16) * 8
    mesh = plsc.VectorSubcoreMesh(core_axis_name="c", subcore_axis_name="s",
                                  num_cores=NC, num_subcores=NS)
    cp = pltpu.CompilerParams()
    if "needs_layout_passes" in pltpu.CompilerParams.__dataclass_fields__:
        cp = dataclasses.replace(cp, needs_layout_passes=False)

    @functools.partial(
        pl.kernel,
        compiler_params=cp,
        out_type=(
            jax.ShapeDtypeStruct((NC, n_acc, H), jnp.float32),
            jax.ShapeDtypeStruct((NW, n_acc), jnp.float32),
        ),
        mesh=mesh,
        scratch_types=[
            pltpu.VMEM((win, CH), jnp.int32),
            pltpu.VMEM((2, CH), jnp.int32),
            pltpu.VMEM((2, CH), jnp.int32),
            pltpu.VMEM((2, CH, H), jnp.float32),
            pltpu.VMEM((n_acc,), jnp.float32),
            pltpu.VMEM_SHARED((n_acc, H), jnp.float32),
            pltpu.SemaphoreType.DMA,
            pltpu.SemaphoreType.DMA,
        ],
    )
    def sc_kernel(x_hbm, pk_hbm, zacc_hbm, zcnt_hbm, acc_hbm, cnt_hbm,
                  pk_v, src_r, dst_r, rows_v, cnt_v, acc_sh, sem0, sem1):
        cid = lax.axis_index("c")
        sid = lax.axis_index("s")
        wid = sid * NC + cid
        nc = jnp.where(cid == 0, nca, ncb)
        base = pl.multiple_of(cid * nca, 8)
        pltpu.sync_copy(zcnt_hbm, cnt_v)
        row0 = sid * rows_per_sub
        pltpu.sync_copy(zacc_hbm, acc_sh.at[pl.ds(row0, rows_per_sub)])
        plsc.subcore_barrier()

        ones_reg = jnp.ones((16,), jnp.float32)
        sems = (sem0, sem1)

        def unpack(j, ring):
            @pl.loop(0, CH // 16)
            def _(k):
                p = pk_v[j, pl.ds(k * 16, 16)]
                d16 = lax.bitwise_and(p, PACK - 1)
                src_r[ring, pl.ds(k * 16, 16)] = lax.shift_right_logical(p, 14)
                dst_r[ring, pl.ds(k * 16, 16)] = d16
                plsc.addupdate_scatter(cnt_v, [d16], ones_reg)

        def fire(buf):
            pltpu.async_copy(x_hbm.at[src_r.at[buf]], rows_v.at[buf],
                             sems[buf])

        def drain(buf):
            pltpu.make_async_copy(x_hbm.at[src_r.at[buf]], rows_v.at[buf],
                                  sems[buf]).wait()
            pltpu.sync_copy(rows_v.at[buf], acc_sh.at[dst_r.at[buf]],
                            add=True)

        def phase(w, n):
            pltpu.sync_copy(pk_hbm.at[sid, pl.ds(base + w * win, win)], pk_v)

            @pl.when(n >= 1)
            def _():
                unpack(0, 0)
                fire(0)

            @pl.when(n >= 2)
            def _():
                unpack(1, 1)
                fire(1)

            @pl.loop(0, n, step=2)
            def _(j):
                drain(0)

                @pl.when(j + 2 < n)
                def _():
                    unpack(j + 2, 0)
                    fire(0)

                @pl.when(j + 1 < n)
                def _():
                    drain(1)

                @pl.when(j + 3 < n)
                def _():
                    unpack(j + 3, 1)
                    fire(1)

        n0 = jnp.minimum(nc, win)
        phase(0, n0)

        @pl.when(nc > win)
        def _():
            phase(1, nc - win)

        plsc.subcore_barrier()
        pltpu.sync_copy(acc_sh.at[pl.ds(row0, rows_per_sub)],
                        acc_hbm.at[cid, pl.ds(row0, rows_per_sub)])
        pltpu.sync_copy(cnt_v, cnt_hbm.at[wid])

    return sc_kernel(x_var, packed3, z_acc, z_cnt)


def _tail_body(acc_ref, cnt_ref, clue_ref, wm_ref, bm_ref, wua_ref, wc_ref,
               bu_ref, g_ref, be_ref, o_ref):
    A = acc_ref[0] + acc_ref[1]
    cnt = jnp.sum(cnt_ref[...], axis=1, keepdims=True)
    m = lax.dot_general(A, wm_ref[...], (((1,), (0,)), ((), ())),
                        precision=lax.Precision.HIGHEST)
    agg = (m + cnt * bm_ref[...]) / (cnt + 1e-6)
    u = lax.dot_general(agg, wua_ref[...], (((1,), (0,)), ((), ())),
                        precision=lax.Precision.HIGHEST)
    u = u + clue_ref[...] * wc_ref[...] + bu_ref[...]
    u = jnp.maximum(u, 0.0)
    mu = jnp.mean(u, axis=1, keepdims=True)
    var = jnp.mean((u - mu) ** 2, axis=1, keepdims=True)
    o_ref[...] = (u - mu) * lax.rsqrt(var + 1e-5) * g_ref[...] + be_ref[...]


def kernel(x_var, edge_index, clue_values, num_con,
           W_msg, b_msg, W_upd, b_upd, gamma, beta):
    N_var, H = x_var.shape
    N_con = clue_values.shape[0]
    E = edge_index.shape[1]
    src = edge_index[0].astype(jnp.int32)
    dst = edge_index[1].astype(jnp.int32)

    packed = src * PACK + dst
    total_chunks = -(-E // CH)
    per_sid = -(-total_chunks // NS)
    cap = per_sid * NS * CH
    if cap > E:
        packed = jnp.concatenate(
            [packed, jnp.full((cap - E,), N_con, jnp.int32)])
    nca = int(round(per_sid * CORE0_FRAC / 8)) * 8
    ncb = per_sid - nca
    ncmax = -(-max(nca, ncb) // 8) * 8
    win = -(-ncmax // 16) * 8
    dim2 = max(per_sid, nca + 2 * win)
    packed3 = packed.reshape(NS, per_sid, CH)
    if dim2 > per_sid:
        packed3 = jnp.concatenate(
            [packed3, jnp.full((NS, dim2 - per_sid, CH), N_con, jnp.int32)],
            axis=1)

    rows_per_sub = (-(-(N_con + 1) // NS) + 7) // 8 * 8
    n_acc = rows_per_sub * NS

    z_acc = jnp.zeros((rows_per_sub, H), jnp.float32)
    z_cnt = jnp.zeros((n_acc,), jnp.float32)

    acc, cnt = _sc_segment_sum(x_var, packed3, z_acc, z_cnt,
                               nca, ncb, n_acc, rows_per_sub, H)
    cnt_t = cnt.T

    delta = (jnp.asarray(num_con) - N_con).astype(jnp.float32)
    beta_eff = (beta + delta).reshape(1, H)

    BLK = 1000
    grid = -(-N_con // BLK)
    out = pl.pallas_call(
        _tail_body,
        grid=(grid,),
        in_specs=[
            pl.BlockSpec((NC, BLK, H), lambda i: (0, i, 0)),
            pl.BlockSpec((BLK, NW), lambda i: (i, 0)),
            pl.BlockSpec((BLK, 1), lambda i: (i, 0)),
            pl.BlockSpec((H, H), lambda i: (0, 0)),
            pl.BlockSpec((1, H), lambda i: (0, 0)),
            pl.BlockSpec((H, H), lambda i: (0, 0)),
            pl.BlockSpec((1, H), lambda i: (0, 0)),
            pl.BlockSpec((1, H), lambda i: (0, 0)),
            pl.BlockSpec((1, H), lambda i: (0, 0)),
            pl.BlockSpec((1, H), lambda i: (0, 0)),
        ],
        out_specs=pl.BlockSpec((BLK, H), lambda i: (i, 0)),
        out_shape=jax.ShapeDtypeStruct((N_con, H), jnp.float32),
    )(acc, cnt_t, clue_values.reshape(N_con, 1), W_msg, b_msg.reshape(1, H),
      W_upd[:H], W_upd[H:H + 1], b_upd.reshape(1, H), gamma.reshape(1, H),
      beta_eff)
    return out


# speedup vs baseline: 1.0734x; 1.0383x over previous
import dataclasses
import functools

import jax
import jax.numpy as jnp
from jax import lax
from jax.experimental import pallas as pl
from jax.experimental.pallas import tpu as pltpu
from jax.experimental.pallas import tpu_sc as plsc

NC = 2
NS = 16
NW = NC * NS
CH = 128
PACK = 1 << 14
CORE0_FRAC = 0.61


def _sc_segment_sum(x_var, packed3, z_acc, z_cnt,
                    nca, ncb, n_acc, rows_per_sub, H):
    ncmax = -(-max(nca, ncb) // 8) * 8
    win = -(-ncmax // 16) * 8
    mesh = plsc.VectorSubcoreMesh(core_axis_name="c", subcore_axis_name="s",
                                  num_cores=NC, num_subcores=NS)
    cp = pltpu.CompilerParams()
    if "needs_layout_passes" in pltpu.CompilerParams.__dataclass_fields__:
        cp = dataclasses.replace(cp, needs_layout_passes=False)

    @functools.partial(
        pl.kernel,
        compiler_params=cp,
        out_type=(
            jax.ShapeDtypeStruct((NC, n_acc, H), jnp.float32),
            jax.ShapeDtypeStruct((NW, n_acc), jnp.float32),
        ),
        mesh=mesh,
        scratch_types=[
            pltpu.VMEM((win, CH), jnp.int32),
            pltpu.VMEM((2, CH), jnp.int32),
            pltpu.VMEM((2, CH), jnp.int32),
            pltpu.VMEM((2, CH, H), jnp.float32),
            pltpu.VMEM((n_acc,), jnp.float32),
            pltpu.VMEM_SHARED((n_acc, H), jnp.float32),
            pltpu.SemaphoreType.DMA,
            pltpu.SemaphoreType.DMA,
        ],
    )
    def sc_kernel(x_hbm, pk_hbm, zacc_hbm, zcnt_hbm, acc_hbm, cnt_hbm,
                  pk_v, src_r, dst_r, rows_v, cnt_v, acc_sh, sem0, sem1):
        cid = lax.axis_index("c")
        sid = lax.axis_index("s")
        wid = sid * NC + cid
        nc = jnp.where(cid == 0, nca, ncb)
        base = pl.multiple_of(cid * nca, 8)
        pltpu.sync_copy(zcnt_hbm, cnt_v)
        row0 = sid * rows_per_sub
        pltpu.sync_copy(zacc_hbm, acc_sh.at[pl.ds(row0, rows_per_sub)])
        plsc.subcore_barrier()

        ones_reg = jnp.ones((16,), jnp.float32)
        sems = (sem0, sem1)

        def unpack(j, ring):
            @pl.loop(0, CH // 16)
            def _(k):
                p = pk_v[j, pl.ds(k * 16, 16)]
                d16 = lax.bitwise_and(p, PACK - 1)
                src_r[ring, pl.ds(k * 16, 16)] = lax.shift_right_logical(p, 14)
                dst_r[ring, pl.ds(k * 16, 16)] = d16
                plsc.addupdate_scatter(cnt_v, [d16], ones_reg)

        def fire(buf):
            pltpu.async_copy(x_hbm.at[src_r.at[buf]], rows_v.at[buf],
                             sems[buf])

        def drain(buf):
            pltpu.make_async_copy(x_hbm.at[src_r.at[buf]], rows_v.at[buf],
                                  sems[buf]).wait()
            pltpu.sync_copy(rows_v.at[buf], acc_sh.at[dst_r.at[buf]],
                            add=True)

        def phase(w, n):
            pltpu.sync_copy(pk_hbm.at[sid, pl.ds(base + w * win, win)], pk_v)

            @pl.when(n >= 1)
            def _():
                unpack(0, 0)
                fire(0)

            @pl.when(n >= 2)
            def _():
                unpack(1, 1)
                fire(1)

            @pl.loop(0, n, step=2)
            def _(j):
                drain(0)

                @pl.when(j + 2 < n)
                def _():
                    unpack(j + 2, 0)
                    fire(0)

                @pl.when(j + 1 < n)
                def _():
                    drain(1)

                @pl.when(j + 3 < n)
                def _():
                    unpack(j + 3, 1)
                    fire(1)

        n0 = jnp.minimum(nc, win)
        phase(0, n0)

        @pl.when(nc > win)
        def _():
            phase(1, nc - win)

        plsc.subcore_barrier()
        pltpu.sync_copy(acc_sh.at[pl.ds(row0, rows_per_sub)],
                        acc_hbm.at[cid, pl.ds(row0, rows_per_sub)])
        pltpu.sync_copy(cnt_v, cnt_hbm.at[wid])

    return sc_kernel(x_var, packed3, z_acc, z_cnt)


def _tail_body(acc_ref, cnt_ref, clue_ref, wm_ref, bm_ref, wua_ref, wc_ref,
               bu_ref, g_ref, be_ref, o_ref):
    A = acc_ref[0] + acc_ref[1]
    cnt = jnp.sum(cnt_ref[...], axis=1, keepdims=True)
    m = lax.dot_general(A, wm_ref[...], (((1,), (0,)), ((), ())),
                        precision=lax.Precision.HIGHEST)
    agg = (m + cnt * bm_ref[...]) / (cnt + 1e-6)
    u = lax.dot_general(agg, wua_ref[...], (((1,), (0,)), ((), ())),
                        precision=lax.Precision.HIGHEST)
    u = u + clue_ref[...] * wc_ref[...] + bu_ref[...]
    u = jnp.maximum(u, 0.0)
    mu = jnp.mean(u, axis=1, keepdims=True)
    var = jnp.mean((u - mu) ** 2, axis=1, keepdims=True)
    o_ref[...] = (u - mu) * lax.rsqrt(var + 1e-5) * g_ref[...] + be_ref[...]


def kernel(x_var, edge_index, clue_values, num_con,
           W_msg, b_msg, W_upd, b_upd, gamma, beta):
    N_var, H = x_var.shape
    N_con = clue_values.shape[0]
    E = edge_index.shape[1]
    src = edge_index[0].astype(jnp.int32)
    dst = edge_index[1].astype(jnp.int32)

    packed = src * PACK + dst
    total_chunks = -(-E // CH)
    per_sid = -(-total_chunks // NS)
    cap = per_sid * NS * CH
    if cap > E:
        packed = jnp.concatenate(
            [packed, jnp.full((cap - E,), N_con, jnp.int32)])
    nca = int(round(per_sid * CORE0_FRAC / 8)) * 8
    ncb = per_sid - nca
    ncmax = -(-max(nca, ncb) // 8) * 8
    win = -(-ncmax // 16) * 8
    dim2 = max(per_sid, nca + 2 * win)
    packed3 = packed.reshape(NS, per_sid, CH)
    if dim2 > per_sid:
        packed3 = jnp.concatenate(
            [packed3, jnp.full((NS, dim2 - per_sid, CH), N_con, jnp.int32)],
            axis=1)

    rows_per_sub = (-(-(N_con + 1) // NS) + 7) // 8 * 8
    n_acc = rows_per_sub * NS

    z_acc = jnp.zeros((rows_per_sub, H), jnp.float32)
    z_cnt = jnp.zeros((n_acc,), jnp.float32)

    acc, cnt = _sc_segment_sum(x_var, packed3, z_acc, z_cnt,
                               nca, ncb, n_acc, rows_per_sub, H)
    cnt_t = cnt.T

    delta = (jnp.asarray(num_con) - N_con).astype(jnp.float32)
    beta_eff = (beta + delta).reshape(1, H)

    BLK = 1000
    grid = -(-N_con // BLK)
    out = pl.pallas_call(
        _tail_body,
        grid=(grid,),
        in_specs=[
            pl.BlockSpec((NC, BLK, H), lambda i: (0, i, 0)),
            pl.BlockSpec((BLK, NW), lambda i: (i, 0)),
            pl.BlockSpec((BLK, 1), lambda i: (i, 0)),
            pl.BlockSpec((H, H), lambda i: (0, 0)),
            pl.BlockSpec((1, H), lambda i: (0, 0)),
            pl.BlockSpec((H, H), lambda i: (0, 0)),
            pl.BlockSpec((1, H), lambda i: (0, 0)),
            pl.BlockSpec((1, H), lambda i: (0, 0)),
            pl.BlockSpec((1, H), lambda i: (0, 0)),
            pl.BlockSpec((1, H), lambda i: (0, 0)),
        ],
        out_specs=pl.BlockSpec((BLK, H), lambda i: (i, 0)),
        out_shape=jax.ShapeDtypeStruct((N_con, H), jnp.float32),
    )(acc, cnt_t, clue_values.reshape(N_con, 1), W_msg, b_msg.reshape(1, H),
      W_upd[:H], W_upd[H:H + 1], b_upd.reshape(1, H), gamma.reshape(1, H),
      beta_eff)
    return out


# speedup vs baseline: 1.1807x; 1.1000x over previous
import dataclasses
import functools

import jax
import jax.numpy as jnp
from jax import lax
from jax.experimental import pallas as pl
from jax.experimental.pallas import tpu as pltpu
from jax.experimental.pallas import tpu_sc as plsc

NC = 2
NS = 16
NW = NC * NS
CH = 128
PACK = 1 << 14
CORE0_FRAC = 0.61


def _sc_segment_sum(x_var, packed3, z_acc, z_cnt,
                    nca, ncb, n_acc, rows_per_sub, H):
    ncmax = -(-max(nca, ncb) // 8) * 8
    win = -(-ncmax // 16) * 8
    mesh = plsc.VectorSubcoreMesh(core_axis_name="c", subcore_axis_name="s",
                                  num_cores=NC, num_subcores=NS)
    cp = pltpu.CompilerParams()
    if "needs_layout_passes" in pltpu.CompilerParams.__dataclass_fields__:
        cp = dataclasses.replace(cp, needs_layout_passes=False)

    @functools.partial(
        pl.kernel,
        compiler_params=cp,
        out_type=(
            jax.ShapeDtypeStruct((NC, n_acc, H), jnp.float32),
            jax.ShapeDtypeStruct((NW, n_acc), jnp.float32),
        ),
        mesh=mesh,
        scratch_types=[
            pltpu.VMEM((win, CH), jnp.int32),
            pltpu.VMEM((2, CH), jnp.int32),
            pltpu.VMEM((2, CH), jnp.int32),
            pltpu.VMEM((2, CH, H), jnp.float32),
            pltpu.VMEM((n_acc,), jnp.float32),
            pltpu.VMEM_SHARED((n_acc, H), jnp.float32),
            pltpu.SemaphoreType.DMA,
            pltpu.SemaphoreType.DMA,
        ],
    )
    def sc_kernel(x_hbm, pk_hbm, zacc_hbm, zcnt_hbm, acc_hbm, cnt_hbm,
                  pk_v, src_r, dst_r, rows_v, cnt_v, acc_sh, sem0, sem1):
        cid = lax.axis_index("c")
        sid = lax.axis_index("s")
        wid = sid * NC + cid
        nc = jnp.where(cid == 0, nca, ncb)
        base = pl.multiple_of(cid * nca, 8)
        pltpu.sync_copy(zcnt_hbm, cnt_v)
        row0 = sid * rows_per_sub
        pltpu.sync_copy(zacc_hbm, acc_sh.at[pl.ds(row0, rows_per_sub)])
        plsc.subcore_barrier()

        ones_reg = jnp.ones((16,), jnp.float32)
        sems = (sem0, sem1)

        def unpack(j, ring):
            @pl.loop(0, CH // 16)
            def _(k):
                p = pk_v[j, pl.ds(k * 16, 16)]
                d16 = lax.bitwise_and(p, PACK - 1)
                src_r[ring, pl.ds(k * 16, 16)] = lax.shift_right_logical(p, 14)
                dst_r[ring, pl.ds(k * 16, 16)] = d16
                plsc.addupdate_scatter(cnt_v, [d16], ones_reg)

        def fire(buf):
            pltpu.async_copy(x_hbm.at[src_r.at[buf]], rows_v.at[buf],
                             sems[buf])

        def drain(buf):
            pltpu.make_async_copy(x_hbm.at[src_r.at[buf]], rows_v.at[buf],
                                  sems[buf]).wait()
            pltpu.sync_copy(rows_v.at[buf], acc_sh.at[dst_r.at[buf]],
                            add=True)

        def phase(w, n):
            pltpu.sync_copy(pk_hbm.at[sid, pl.ds(base + w * win, win)], pk_v)

            @pl.when(n >= 1)
            def _():
                unpack(0, 0)
                fire(0)

            @pl.when(n >= 2)
            def _():
                unpack(1, 1)
                fire(1)

            @pl.loop(0, n, step=2)
            def _(j):
                drain(0)

                @pl.when(j + 2 < n)
                def _():
                    unpack(j + 2, 0)
                    fire(0)

                @pl.when(j + 1 < n)
                def _():
                    drain(1)

                @pl.when(j + 3 < n)
                def _():
                    unpack(j + 3, 1)
                    fire(1)

        n0 = jnp.minimum(nc, win)
        phase(0, n0)

        @pl.when(nc > win)
        def _():
            phase(1, nc - win)

        plsc.subcore_barrier()
        pltpu.sync_copy(acc_sh.at[pl.ds(row0, rows_per_sub)],
                        acc_hbm.at[cid, pl.ds(row0, rows_per_sub)])
        pltpu.sync_copy(cnt_v, cnt_hbm.at[wid])

    return sc_kernel(x_var, packed3, z_acc, z_cnt)


def _tail_body(acc_ref, cnt_ref, clue_ref, wm_ref, bm_ref, wua_ref, wc_ref,
               bu_ref, g_ref, be_ref, o_ref):
    A = acc_ref[0] + acc_ref[1]
    cnt = jnp.sum(cnt_ref[...], axis=1, keepdims=True)
    m = lax.dot_general(A, wm_ref[...], (((1,), (0,)), ((), ())),
                        precision=lax.Precision.DEFAULT)
    agg = (m + cnt * bm_ref[...]) / (cnt + 1e-6)
    u = lax.dot_general(agg, wua_ref[...], (((1,), (0,)), ((), ())),
                        precision=lax.Precision.DEFAULT)
    u = u + clue_ref[...] * wc_ref[...] + bu_ref[...]
    u = jnp.maximum(u, 0.0)
    mu = jnp.mean(u, axis=1, keepdims=True)
    var = jnp.mean((u - mu) ** 2, axis=1, keepdims=True)
    o_ref[...] = (u - mu) * lax.rsqrt(var + 1e-5) * g_ref[...] + be_ref[...]


def kernel(x_var, edge_index, clue_values, num_con,
           W_msg, b_msg, W_upd, b_upd, gamma, beta):
    N_var, H = x_var.shape
    N_con = clue_values.shape[0]
    E = edge_index.shape[1]
    src = edge_index[0].astype(jnp.int32)
    dst = edge_index[1].astype(jnp.int32)

    packed = src * PACK + dst
    total_chunks = -(-E // CH)
    per_sid = -(-total_chunks // NS)
    cap = per_sid * NS * CH
    if cap > E:
        packed = jnp.concatenate(
            [packed, jnp.full((cap - E,), N_con, jnp.int32)])
    nca = int(round(per_sid * CORE0_FRAC / 8)) * 8
    ncb = per_sid - nca
    ncmax = -(-max(nca, ncb) // 8) * 8
    win = -(-ncmax // 16) * 8
    dim2 = max(per_sid, nca + 2 * win)
    packed3 = packed.reshape(NS, per_sid, CH)
    if dim2 > per_sid:
        packed3 = jnp.concatenate(
            [packed3, jnp.full((NS, dim2 - per_sid, CH), N_con, jnp.int32)],
            axis=1)

    rows_per_sub = (-(-(N_con + 1) // NS) + 7) // 8 * 8
    n_acc = rows_per_sub * NS

    z_acc = jnp.zeros((rows_per_sub, H), jnp.float32)
    z_cnt = jnp.zeros((n_acc,), jnp.float32)

    acc, cnt = _sc_segment_sum(x_var, packed3, z_acc, z_cnt,
                               nca, ncb, n_acc, rows_per_sub, H)
    cnt_t = cnt.T

    delta = (jnp.asarray(num_con) - N_con).astype(jnp.float32)
    beta_eff = (beta + delta).reshape(1, H)

    BLK = 1000
    grid = -(-N_con // BLK)
    out = pl.pallas_call(
        _tail_body,
        grid=(grid,),
        in_specs=[
            pl.BlockSpec((NC, BLK, H), lambda i: (0, i, 0)),
            pl.BlockSpec((BLK, NW), lambda i: (i, 0)),
            pl.BlockSpec((BLK, 1), lambda i: (i, 0)),
            pl.BlockSpec((H, H), lambda i: (0, 0)),
            pl.BlockSpec((1, H), lambda i: (0, 0)),
            pl.BlockSpec((H, H), lambda i: (0, 0)),
            pl.BlockSpec((1, H), lambda i: (0, 0)),
            pl.BlockSpec((1, H), lambda i: (0, 0)),
            pl.BlockSpec((1, H), lambda i: (0, 0)),
            pl.BlockSpec((1, H), lambda i: (0, 0)),
        ],
        out_specs=pl.BlockSpec((BLK, H), lambda i: (i, 0)),
        out_shape=jax.ShapeDtypeStruct((N_con, H), jnp.float32),
    )(acc, cnt_t, clue_values.reshape(N_con, 1), W_msg, b_msg.reshape(1, H),
      W_upd[:H], W_upd[H:H + 1], b_upd.reshape(1, H), gamma.reshape(1, H),
      beta_eff)
    return out


# speedup vs baseline: 1.1924x; 1.0099x over previous
import dataclasses
import functools

import jax
import jax.numpy as jnp
from jax import lax
from jax.experimental import pallas as pl
from jax.experimental.pallas import tpu as pltpu
from jax.experimental.pallas import tpu_sc as plsc

NC = 2
NS = 16
NW = NC * NS
CH = 128
PACK = 1 << 14
CORE0_FRAC = 0.61


def _sc_segment_sum(x_var, packed3, z_acc, z_cnt,
                    nca, ncb, n_acc, rows_per_sub, H):
    ncmax = -(-max(nca, ncb) // 8) * 8
    win = -(-ncmax // 16) * 8
    mesh = plsc.VectorSubcoreMesh(core_axis_name="c", subcore_axis_name="s",
                                  num_cores=NC, num_subcores=NS)
    cp = pltpu.CompilerParams()
    if "needs_layout_passes" in pltpu.CompilerParams.__dataclass_fields__:
        cp = dataclasses.replace(cp, needs_layout_passes=False)

    @functools.partial(
        pl.kernel,
        compiler_params=cp,
        out_type=(
            jax.ShapeDtypeStruct((NC, n_acc, H), jnp.float32),
            jax.ShapeDtypeStruct((NW, n_acc), jnp.float32),
        ),
        mesh=mesh,
        scratch_types=[
            pltpu.VMEM((win, CH), jnp.int32),
            pltpu.VMEM((2, CH), jnp.int32),
            pltpu.VMEM((2, CH), jnp.int32),
            pltpu.VMEM((2, CH, H), jnp.float32),
            pltpu.VMEM((n_acc,), jnp.float32),
            pltpu.VMEM_SHARED((n_acc, H), jnp.float32),
            pltpu.SemaphoreType.DMA,
            pltpu.SemaphoreType.DMA,
        ],
    )
    def sc_kernel(x_hbm, pk_hbm, zacc_hbm, zcnt_hbm, acc_hbm, cnt_hbm,
                  pk_v, src_r, dst_r, rows_v, cnt_v, acc_sh, sem0, sem1):
        cid = lax.axis_index("c")
        sid = lax.axis_index("s")
        wid = sid * NC + cid
        nc = jnp.where(cid == 0, nca, ncb)
        base = pl.multiple_of(cid * nca, 8)
        pltpu.sync_copy(zcnt_hbm, cnt_v)
        row0 = sid * rows_per_sub
        pltpu.sync_copy(zacc_hbm, acc_sh.at[pl.ds(row0, rows_per_sub)])
        plsc.subcore_barrier()

        ones_reg = jnp.ones((16,), jnp.float32)
        sems = (sem0, sem1)

        def unpack(j, ring):
            @pl.loop(0, CH // 16)
            def _(k):
                p = pk_v[j, pl.ds(k * 16, 16)]
                d16 = lax.bitwise_and(p, PACK - 1)
                src_r[ring, pl.ds(k * 16, 16)] = lax.shift_right_logical(p, 14)
                dst_r[ring, pl.ds(k * 16, 16)] = d16
                plsc.addupdate_scatter(cnt_v, [d16], ones_reg)

        def fire(buf):
            pltpu.async_copy(x_hbm.at[src_r.at[buf]], rows_v.at[buf],
                             sems[buf])

        def drain(buf):
            pltpu.make_async_copy(x_hbm.at[src_r.at[buf]], rows_v.at[buf],
                                  sems[buf]).wait()
            pltpu.sync_copy(rows_v.at[buf], acc_sh.at[dst_r.at[buf]],
                            add=True)

        def phase(w, n):
            pltpu.sync_copy(pk_hbm.at[sid, pl.ds(base + w * win, win)], pk_v)

            @pl.when(n >= 1)
            def _():
                unpack(0, 0)
                fire(0)

            @pl.when(n >= 2)
            def _():
                unpack(1, 1)
                fire(1)

            @pl.loop(0, n, step=2)
            def _(j):
                drain(0)

                @pl.when(j + 2 < n)
                def _():
                    unpack(j + 2, 0)
                    fire(0)

                @pl.when(j + 1 < n)
                def _():
                    drain(1)

                @pl.when(j + 3 < n)
                def _():
                    unpack(j + 3, 1)
                    fire(1)

        n0 = jnp.minimum(nc, win)
        phase(0, n0)

        @pl.when(nc > win)
        def _():
            phase(1, nc - win)

        plsc.subcore_barrier()
        pltpu.sync_copy(acc_sh.at[pl.ds(row0, rows_per_sub)],
                        acc_hbm.at[cid, pl.ds(row0, rows_per_sub)])
        pltpu.sync_copy(cnt_v, cnt_hbm.at[wid])

    return sc_kernel(x_var, packed3, z_acc, z_cnt)


def _tail_body(acc_ref, cnt_ref, clue_ref, wm_ref, bm_ref, wua_ref, wc_ref,
               bu_ref, g_ref, be_ref, o_ref):
    A = acc_ref[0] + acc_ref[1]
    cnt = jnp.sum(cnt_ref[...], axis=1, keepdims=True)
    m = lax.dot_general(A, wm_ref[...], (((1,), (0,)), ((), ())),
                        precision=lax.Precision.DEFAULT)
    agg = (m + cnt * bm_ref[...]) / (cnt + 1e-6)
    u = lax.dot_general(agg, wua_ref[...], (((1,), (0,)), ((), ())),
                        precision=lax.Precision.DEFAULT)
    u = u + clue_ref[...] * wc_ref[...] + bu_ref[...]
    u = jnp.maximum(u, 0.0)
    mu = jnp.mean(u, axis=1, keepdims=True)
    var = jnp.mean((u - mu) ** 2, axis=1, keepdims=True)
    o_ref[...] = (u - mu) * lax.rsqrt(var + 1e-5) * g_ref[...] + be_ref[...]


def kernel(x_var, edge_index, clue_values, num_con,
           W_msg, b_msg, W_upd, b_upd, gamma, beta):
    N_var, H = x_var.shape
    N_con = clue_values.shape[0]
    E = edge_index.shape[1]
    src = edge_index[0].astype(jnp.int32)
    dst = edge_index[1].astype(jnp.int32)

    packed = src * PACK + dst
    total_chunks = -(-E // CH)
    per_sid = -(-total_chunks // NS)
    cap = per_sid * NS * CH
    if cap > E:
        packed = jnp.concatenate(
            [packed, jnp.full((cap - E,), N_con, jnp.int32)])
    nca = int(round(per_sid * CORE0_FRAC / 8)) * 8
    ncb = per_sid - nca
    ncmax = -(-max(nca, ncb) // 8) * 8
    win = -(-ncmax // 16) * 8
    dim2 = max(per_sid, nca + 2 * win)
    packed3 = packed.reshape(NS, per_sid, CH)
    if dim2 > per_sid:
        packed3 = jnp.concatenate(
            [packed3, jnp.full((NS, dim2 - per_sid, CH), N_con, jnp.int32)],
            axis=1)

    rows_per_sub = (-(-(N_con + 1) // NS) + 7) // 8 * 8
    n_acc = rows_per_sub * NS

    z_acc = jnp.zeros((rows_per_sub, H), jnp.float32)
    z_cnt = jnp.zeros((n_acc,), jnp.float32)

    acc, cnt = _sc_segment_sum(x_var, packed3, z_acc, z_cnt,
                               nca, ncb, n_acc, rows_per_sub, H)
    cnt_t = cnt.T

    delta = (jnp.asarray(num_con) - N_con).astype(jnp.float32)
    beta_eff = (beta + delta).reshape(1, H)

    BLK = 2000
    grid = -(-N_con // BLK)
    out = pl.pallas_call(
        _tail_body,
        grid=(grid,),
        in_specs=[
            pl.BlockSpec((NC, BLK, H), lambda i: (0, i, 0)),
            pl.BlockSpec((BLK, NW), lambda i: (i, 0)),
            pl.BlockSpec((BLK, 1), lambda i: (i, 0)),
            pl.BlockSpec((H, H), lambda i: (0, 0)),
            pl.BlockSpec((1, H), lambda i: (0, 0)),
            pl.BlockSpec((H, H), lambda i: (0, 0)),
            pl.BlockSpec((1, H), lambda i: (0, 0)),
            pl.BlockSpec((1, H), lambda i: (0, 0)),
            pl.BlockSpec((1, H), lambda i: (0, 0)),
            pl.BlockSpec((1, H), lambda i: (0, 0)),
        ],
        out_specs=pl.BlockSpec((BLK, H), lambda i: (i, 0)),
        out_shape=jax.ShapeDtypeStruct((N_con, H), jnp.float32),
    )(acc, cnt_t, clue_values.reshape(N_con, 1), W_msg, b_msg.reshape(1, H),
      W_upd[:H], W_upd[H:H + 1], b_upd.reshape(1, H), gamma.reshape(1, H),
      beta_eff)
    return out
